# baseline (speedup 1.0000x reference)
import functools

import jax
import jax.numpy as jnp
from jax import lax
from jax.experimental import pallas as pl
from jax.experimental.pallas import tpu as pltpu
from jax.experimental.pallas import tpu_sc as plsc

N_FIELDS = 26
VOCAB = 100000
DIM = 32
B = 1024
T = 50

N_ROWS = N_FIELDS * B * T
NW = 32
PER_W = N_ROWS // NW
IDX_W = 128
KG = 13
GROUP = KG * IDX_W
NGROUPS = PER_W // GROUP
assert NGROUPS * GROUP == PER_W

_mesh = plsc.VectorSubcoreMesh(core_axis_name="c", subcore_axis_name="s")


@functools.partial(
    pl.kernel,
    mesh=_mesh,
    out_type=jax.ShapeDtypeStruct((N_ROWS, DIM), jnp.float32),
    scratch_types=[
        pltpu.VMEM((GROUP,), jnp.int32),
        pltpu.VMEM((GROUP, DIM), jnp.float32),
        pltpu.SemaphoreType.DMA,
    ],
    compiler_params=pltpu.CompilerParams(use_tc_tiling_on_sc=False),
)
def _sc_gather(idx_hbm, tab_hbm, out_hbm, idx_v, rows_v, gsem):
    cid = lax.axis_index("c")
    sid = lax.axis_index("s")
    wid = sid * 2 + cid
    base_row = wid * PER_W

    def group_body(g, carry):
        gbase = base_row + g * GROUP
        pltpu.sync_copy(idx_hbm.at[pl.ds(gbase, GROUP)], idx_v)
        copies = []
        for j in range(KG):
            copies.append(
                pltpu.async_copy(
                    tab_hbm.at[idx_v.at[pl.ds(j * IDX_W, IDX_W)]],
                    rows_v.at[pl.ds(j * IDX_W, IDX_W)],
                    gsem,
                )
            )
        for cp in copies:
            cp.wait()
        pltpu.sync_copy(rows_v, out_hbm.at[pl.ds(gbase, GROUP)])
        return carry

    lax.fori_loop(0, NGROUPS, group_body, 0)


def kernel(x, tables):
    offs = (jnp.arange(N_FIELDS, dtype=jnp.int32) * VOCAB)[:, None, None]
    flat_idx = (x.transpose(2, 0, 1) + offs).reshape(N_ROWS)
    tab = tables.reshape(N_FIELDS * VOCAB, DIM)
    out = _sc_gather(flat_idx, tab)
    return out.reshape(N_FIELDS, B, T, DIM)


# speedup vs baseline: 2.2393x; 2.2393x over previous
import functools

import jax
import jax.numpy as jnp
from jax import lax
from jax.experimental import pallas as pl
from jax.experimental.pallas import tpu as pltpu
from jax.experimental.pallas import tpu_sc as plsc

N_FIELDS = 26
VOCAB = 100000
DIM = 32
B = 1024
T = 50

BLK_B = 128
NBLK_B = B // BLK_B
N_BLOCKS = N_FIELDS * T * NBLK_B
NW = 32
PER_W = N_BLOCKS // NW
N_OUT = N_FIELDS * T * DIM * B
assert PER_W * NW == N_BLOCKS

_mesh = plsc.VectorSubcoreMesh(core_axis_name="c", subcore_axis_name="s")


@functools.partial(
    pl.kernel,
    mesh=_mesh,
    out_type=jax.ShapeDtypeStruct((N_OUT,), jnp.float32),
    scratch_types=[
        pltpu.VMEM((BLK_B,), jnp.int32),
        pltpu.VMEM((BLK_B, DIM), jnp.float32),
        pltpu.VMEM((DIM * BLK_B,), jnp.float32),
        pltpu.SemaphoreType.DMA,
        pltpu.SemaphoreType.DMA,
    ],
    compiler_params=pltpu.CompilerParams(
        use_tc_tiling_on_sc=False, needs_layout_passes=False
    ),
)
def _sc_gather(idx_hbm, tab_hbm, out_hbm, idx_v, gath_v, tr_v, gsem, osem):
    cid = lax.axis_index("c")
    sid = lax.axis_index("s")
    wid = sid * 2 + cid
    base_blk = wid * PER_W

    scat = [lax.iota(jnp.int32, 16) * BLK_B + h * 16 * BLK_B for h in range(2)]

    def block_body(k, carry):
        u = base_blk + k
        ft = u // NBLK_B
        bc = u % NBLK_B
        out_base = ft * (DIM * B) + bc * BLK_B
        pltpu.sync_copy(idx_hbm.at[pl.ds(u * BLK_B, BLK_B)], idx_v)
        pltpu.async_copy(tab_hbm.at[idx_v], gath_v, gsem).wait()
        for j in range(BLK_B):
            for h in range(2):
                vals = gath_v[j, pl.ds(h * 16, 16)]
                plsc.store_scatter(tr_v, [scat[h] + j], vals)
        copies = [
            pltpu.async_copy(
                tr_v.at[pl.ds(d * BLK_B, BLK_B)],
                out_hbm.at[pl.ds(out_base + d * B, BLK_B)],
                osem,
            )
            for d in range(DIM)
        ]
        for cp in copies:
            cp.wait()
        return carry

    lax.fori_loop(0, PER_W, block_body, 0)


def kernel(x, tables):
    offs = (jnp.arange(N_FIELDS, dtype=jnp.int32) * VOCAB)[:, None, None]
    flat_idx = (x.transpose(2, 1, 0) + offs).reshape(N_FIELDS * T * B)
    tab = tables.reshape(N_FIELDS * VOCAB, DIM)
    out = _sc_gather(flat_idx, tab)
    out = out.reshape(N_FIELDS, T, DIM, B)
    return out.transpose(0, 3, 1, 2)


# speedup vs baseline: 2.9531x; 1.3187x over previous
import functools

import jax
import jax.numpy as jnp
from jax import lax
from jax.experimental import pallas as pl
from jax.experimental.pallas import tpu as pltpu
from jax.experimental.pallas import tpu_sc as plsc

N_FIELDS = 26
VOCAB = 100000
DIM = 32
B = 1024
T = 50

BLK_B = 128
NBLK_B = B // BLK_B
N_BLOCKS = N_FIELDS * T * NBLK_B
NW = 32
PER_W = N_BLOCKS // NW
N_PAIR = (PER_W + 2) // 2
N_OUT = N_FIELDS * T * DIM * B
BLK_W = DIM * BLK_B
assert PER_W * NW == N_BLOCKS

_mesh = plsc.VectorSubcoreMesh(core_axis_name="c", subcore_axis_name="s")


@functools.partial(
    pl.kernel,
    mesh=_mesh,
    out_type=jax.ShapeDtypeStruct((N_OUT,), jnp.float32),
    scratch_types=[
        pltpu.VMEM((BLK_B,), jnp.int32),
        pltpu.VMEM((BLK_B,), jnp.int32),
        pltpu.VMEM((BLK_B, DIM), jnp.float32),
        pltpu.VMEM((BLK_B, DIM), jnp.float32),
        pltpu.VMEM((BLK_W,), jnp.float32),
        pltpu.VMEM((BLK_W,), jnp.float32),
        pltpu.SemaphoreType.DMA,
        pltpu.SemaphoreType.DMA,
        pltpu.SemaphoreType.DMA,
        pltpu.SemaphoreType.DMA,
        pltpu.SemaphoreType.DMA,
        pltpu.SemaphoreType.DMA,
    ],
    compiler_params=pltpu.CompilerParams(
        use_tc_tiling_on_sc=False, needs_layout_passes=False
    ),
)
def _sc_gather(
    idx_hbm, tab_hbm, out_hbm,
    idx_a, idx_b, gath_a, gath_b, tr_a, tr_b,
    isem_a, isem_b, gsem_a, gsem_b, osem_a, osem_b,
):
    cid = lax.axis_index("c")
    sid = lax.axis_index("s")
    wid = sid * 2 + cid
    base = wid * PER_W
    last = base + PER_W - 1

    idx_v = [idx_a, idx_b]
    gath_v = [gath_a, gath_b]
    tr_v = [tr_a, tr_b]
    isem = [isem_a, isem_b]
    gsem = [gsem_a, gsem_b]
    osem = [osem_a, osem_b]

    scat = [lax.iota(jnp.int32, 16) * BLK_B + h * 16 * BLK_B for h in range(2)]

    def do_slot(m, s, j):
        u = jnp.minimum(base + s, last)
        up = jnp.minimum(base + s + 2, last)
        pltpu.make_async_copy(idx_hbm.at[pl.ds(0, BLK_B)], idx_v[j ^ 1],
                              isem[j ^ 1]).wait()
        pltpu.async_copy(tab_hbm.at[idx_v[j ^ 1]], gath_v[j ^ 1], gsem[j ^ 1])
        pltpu.make_async_copy(tab_hbm.at[pl.ds(0, BLK_B), :], gath_v[j],
                              gsem[j]).wait()
        pltpu.async_copy(idx_hbm.at[pl.ds(up * BLK_B, BLK_B)], idx_v[j],
                         isem[j])
        @pl.when(m > 0)
        def _():
            pltpu.make_async_copy(out_hbm.at[pl.ds(0, BLK_W)], tr_v[j],
                                  osem[j]).wait()
        for row in range(BLK_B):
            for h in range(2):
                vals = gath_v[j][row, pl.ds(h * 16, 16)]
                plsc.store_scatter(tr_v[j], [scat[h] + row], vals)
        ft = u // NBLK_B
        bc = u % NBLK_B
        out_base = ft * (DIM * B) + bc * (8 * BLK_B)
        for dt in range(4):
            pltpu.async_copy(
                tr_v[j].at[pl.ds(dt * 8 * BLK_B, 8 * BLK_B)],
                out_hbm.at[pl.ds(out_base + dt * (8 * B), 8 * BLK_B)],
                osem[j],
            )

    def pair_body(m, carry):
        do_slot(m, 2 * m, 0)
        do_slot(m, 2 * m + 1, 1)
        return carry

    pltpu.sync_copy(idx_hbm.at[pl.ds(base * BLK_B, BLK_B)], idx_v[0])
    pltpu.async_copy(tab_hbm.at[idx_v[0]], gath_v[0], gsem[0])
    pltpu.async_copy(idx_hbm.at[pl.ds((base + 1) * BLK_B, BLK_B)], idx_v[1],
                     isem[1])
    lax.fori_loop(0, N_PAIR, pair_body, 0)
    pltpu.make_async_copy(tab_hbm.at[pl.ds(0, BLK_B), :], gath_v[0],
                          gsem[0]).wait()
    pltpu.make_async_copy(idx_hbm.at[pl.ds(0, BLK_B)], idx_v[1],
                          isem[1]).wait()
    pltpu.make_async_copy(out_hbm.at[pl.ds(0, BLK_W)], tr_v[0], osem[0]).wait()
    pltpu.make_async_copy(out_hbm.at[pl.ds(0, BLK_W)], tr_v[1], osem[1]).wait()


def kernel(x, tables):
    offs = (jnp.arange(N_FIELDS, dtype=jnp.int32) * VOCAB)[:, None, None]
    flat_idx = (x.transpose(2, 1, 0) + offs).reshape(N_FIELDS * T * B)
    tab = tables.reshape(N_FIELDS * VOCAB, DIM)
    out = _sc_gather(flat_idx, tab)
    out = out.reshape(N_FIELDS, T, DIM // 8, B // BLK_B, 8, BLK_B)
    out = out.transpose(0, 3, 5, 1, 2, 4).reshape(N_FIELDS, B, T, DIM)
    return out


# speedup vs baseline: 3.1162x; 1.0552x over previous
import functools

import jax
import jax.numpy as jnp
from jax import lax
from jax.experimental import pallas as pl
from jax.experimental.pallas import tpu as pltpu
from jax.experimental.pallas import tpu_sc as plsc

N_FIELDS = 26
VOCAB = 100000
DIM = 32
B = 1024
T = 50

BLK_B = 128
NBLK_B = B // BLK_B
N_BLOCKS = N_FIELDS * T * NBLK_B
NW = 32
PER_W = N_BLOCKS // NW
N_PAIR = (PER_W + 2) // 2
N_OUT = N_FIELDS * T * DIM * B
BLK_W = DIM * BLK_B
assert PER_W * NW == N_BLOCKS

_mesh = plsc.VectorSubcoreMesh(core_axis_name="c", subcore_axis_name="s")


@functools.partial(
    pl.kernel,
    mesh=_mesh,
    out_type=jax.ShapeDtypeStruct((N_OUT,), jnp.float32),
    scratch_types=[
        pltpu.VMEM((BLK_B,), jnp.int32),
        pltpu.VMEM((BLK_B,), jnp.int32),
        pltpu.VMEM((BLK_B, DIM), jnp.float32),
        pltpu.VMEM((BLK_B, DIM), jnp.float32),
        pltpu.VMEM((BLK_W,), jnp.float32),
        pltpu.VMEM((BLK_W,), jnp.float32),
        pltpu.SemaphoreType.DMA,
        pltpu.SemaphoreType.DMA,
        pltpu.SemaphoreType.DMA,
        pltpu.SemaphoreType.DMA,
        pltpu.SemaphoreType.DMA,
        pltpu.SemaphoreType.DMA,
    ],
    compiler_params=pltpu.CompilerParams(
        use_tc_tiling_on_sc=False, needs_layout_passes=False
    ),
)
def _sc_gather(
    idx_hbm, tab_hbm, out_hbm,
    idx_a, idx_b, gath_a, gath_b, tr_a, tr_b,
    isem_a, isem_b, gsem_a, gsem_b, osem_a, osem_b,
):
    cid = lax.axis_index("c")
    sid = lax.axis_index("s")
    wid = sid * 2 + cid
    base = wid * PER_W
    last = base + PER_W - 1

    idx_v = [idx_a, idx_b]
    gath_v = [gath_a, gath_b]
    tr_v = [tr_a, tr_b]
    isem = [isem_a, isem_b]
    gsem = [gsem_a, gsem_b]
    osem = [osem_a, osem_b]

    scat = [lax.iota(jnp.int32, 16) * BLK_B + h * 16 * BLK_B for h in range(2)]

    def do_slot(m, s, j):
        u = jnp.minimum(base + s, last)
        un = jnp.minimum(base + s + 1, last)
        up = jnp.minimum(base + s + 2, last)
        pltpu.make_async_copy(idx_hbm.at[pl.ds(0, BLK_B)], idx_v[j ^ 1],
                              isem[j ^ 1]).wait()
        fn = un // (T * NBLK_B)
        pltpu.async_copy(tab_hbm.at[fn].at[idx_v[j ^ 1]], gath_v[j ^ 1],
                         gsem[j ^ 1])
        pltpu.make_async_copy(tab_hbm.at[0].at[pl.ds(0, BLK_B), :], gath_v[j],
                              gsem[j]).wait()
        pltpu.async_copy(idx_hbm.at[pl.ds(up * BLK_B, BLK_B)], idx_v[j],
                         isem[j])
        @pl.when(m > 0)
        def _():
            pltpu.make_async_copy(out_hbm.at[pl.ds(0, BLK_W)], tr_v[j],
                                  osem[j]).wait()
        prev = None
        for row in range(BLK_B):
            cur = [gath_v[j][row, pl.ds(h * 16, 16)] for h in range(2)]
            if prev is not None:
                for h in range(2):
                    plsc.store_scatter(tr_v[j], [scat[h] + (row - 1)], prev[h])
            prev = cur
        for h in range(2):
            plsc.store_scatter(tr_v[j], [scat[h] + (BLK_B - 1)], prev[h])
        ft = u // NBLK_B
        bc = u % NBLK_B
        out_base = ft * (DIM * B) + bc * (8 * BLK_B)
        for dt in range(4):
            pltpu.async_copy(
                tr_v[j].at[pl.ds(dt * 8 * BLK_B, 8 * BLK_B)],
                out_hbm.at[pl.ds(out_base + dt * (8 * B), 8 * BLK_B)],
                osem[j],
            )

    def pair_body(m, carry):
        do_slot(m, 2 * m, 0)
        do_slot(m, 2 * m + 1, 1)
        return carry

    pltpu.sync_copy(idx_hbm.at[pl.ds(base * BLK_B, BLK_B)], idx_v[0])
    f0 = base // (T * NBLK_B)
    pltpu.async_copy(tab_hbm.at[f0].at[idx_v[0]], gath_v[0], gsem[0])
    pltpu.async_copy(idx_hbm.at[pl.ds((base + 1) * BLK_B, BLK_B)], idx_v[1],
                     isem[1])
    lax.fori_loop(0, N_PAIR, pair_body, 0)
    pltpu.make_async_copy(tab_hbm.at[0].at[pl.ds(0, BLK_B), :], gath_v[0],
                          gsem[0]).wait()
    pltpu.make_async_copy(idx_hbm.at[pl.ds(0, BLK_B)], idx_v[1],
                          isem[1]).wait()
    pltpu.make_async_copy(out_hbm.at[pl.ds(0, BLK_W)], tr_v[0], osem[0]).wait()
    pltpu.make_async_copy(out_hbm.at[pl.ds(0, BLK_W)], tr_v[1], osem[1]).wait()


def kernel(x, tables):
    flat_idx = x.transpose(2, 1, 0).reshape(N_FIELDS * T * B)
    out = _sc_gather(flat_idx, tables)
    out = out.reshape(N_FIELDS, T, DIM // 8, B // BLK_B, 8, BLK_B)
    out = out.transpose(0, 3, 5, 1, 2, 4).reshape(N_FIELDS, B, T, DIM)
    return out
